# baseline (speedup 1.0000x reference)
import functools

import jax
import jax.numpy as jnp
from jax import lax
from jax.experimental import pallas as pl
from jax.experimental.pallas import tpu as pltpu
from jax.experimental.pallas import tpu_sc as plsc

T = 2048
D = 768
O = 768
E = 8
BT = 128
NCHUNK = T // BT + E - 1
NSPLIT = 3
OH = O // NSPLIT
TP = NCHUNK * BT
NCORE = 2
NSUB = 16
NW = NCORE * NSUB
TPW = T // NW


def _route_body(x1_ref, x2_ref, wg_ref, dest_ref, ce_ref, loss_ref):
    xf = jnp.concatenate([x1_ref[...], x2_ref[...]], axis=1)
    wg = wg_ref[...]
    logits = lax.dot_general(xf, wg, (((1,), (0,)), ((), ())),
                             preferred_element_type=jnp.float32)
    lt = logits.T
    m = jnp.max(lt, axis=0, keepdims=True)
    eiota = lax.broadcasted_iota(jnp.int32, (E, T), 0)
    e = jnp.min(jnp.where(lt == m, eiota, E), axis=0, keepdims=True)
    oh = (eiota == e).astype(jnp.float32)
    counts = jnp.sum(oh, axis=1, keepdims=True)

    c = oh
    k = 1
    while k < T:
        c = c + jnp.concatenate(
            [jnp.zeros((E, k), jnp.float32), c[:, : T - k]], axis=1)
        k *= 2
    rank = jnp.sum(c * oh, axis=0, keepdims=True) - 1.0

    nch = jnp.floor((counts + float(BT - 1)) * (1.0 / BT))
    inc = nch
    k = 1
    while k < E:
        inc = inc + jnp.concatenate(
            [jnp.zeros((k, 1), jnp.float32), inc[: E - k, :]], axis=0)
        k *= 2
    po = (inc - nch) * float(BT)
    off_t = jnp.sum(po * oh, axis=0, keepdims=True)
    dest_ref[...] = (off_t + rank).astype(jnp.int32)

    ciota = lax.broadcasted_iota(
        jnp.int32, (E, NCHUNK), 1).astype(jnp.float32) * float(BT)
    ind = (po <= ciota).astype(jnp.int32)
    ce_ref[...] = jnp.sum(ind, axis=0, keepdims=True) - 1

    dev = counts - float(T // E)
    s = jnp.sum(dev * dev, keepdims=True)
    loss_ref[...] = s * (0.02 / float(E - 1) / float((T // E) ** 2))


def _route(xf, w_gate):
    return pl.pallas_call(
        _route_body,
        grid=(1,),
        in_specs=[
            pl.BlockSpec((T, D // 2), lambda i: (0, 0)),
            pl.BlockSpec((T, D // 2), lambda i: (0, 1)),
            pl.BlockSpec((D, E), lambda i: (0, 0)),
        ],
        out_specs=(
            pl.BlockSpec((1, T), lambda i: (0, 0)),
            pl.BlockSpec((1, NCHUNK), lambda i: (0, 0)),
            pl.BlockSpec((1, 1), lambda i: (0, 0)),
        ),
        out_shape=(
            jax.ShapeDtypeStruct((1, T), jnp.int32),
            jax.ShapeDtypeStruct((1, NCHUNK), jnp.int32),
            jax.ShapeDtypeStruct((1, 1), jnp.float32),
        ),
    )(xf, xf, w_gate)


@functools.cache
def _sc_kernels():
    mesh = plsc.VectorSubcoreMesh(core_axis_name="c", subcore_axis_name="s")

    @functools.partial(
        pl.kernel,
        mesh=mesh,
        out_type=jax.ShapeDtypeStruct((TP, D), jnp.float32),
        scratch_types=[
            pltpu.VMEM((TPW,), jnp.int32),
            pltpu.VMEM((TPW, D), jnp.float32),
            pltpu.SemaphoreType.DMA,
        ],
    )
    def _dispatch(x_hbm, dest_hbm, xs_hbm, idx_v, rows_v, sem):
        wid = lax.axis_index("s") * NCORE + lax.axis_index("c")
        base = wid * TPW
        pltpu.sync_copy(dest_hbm.at[pl.ds(base, TPW)], idx_v)
        pltpu.sync_copy(x_hbm.at[pl.ds(base, TPW)], rows_v)
        pltpu.async_copy(rows_v, xs_hbm.at[idx_v], sem).wait()

    @functools.partial(
        pl.kernel,
        mesh=mesh,
        out_type=jax.ShapeDtypeStruct((T, O), jnp.float32),
        scratch_types=[
            pltpu.VMEM((TPW,), jnp.int32),
            *[pltpu.VMEM((TPW, OH), jnp.float32) for _ in range(NSPLIT)],
            *[pltpu.SemaphoreType.DMA for _ in range(NSPLIT)],
        ],
    )
    def _combine(*refs):
        ys_hbm = refs[:NSPLIT]
        dest_hbm, y_hbm, idx_v = refs[NSPLIT:NSPLIT + 3]
        rows = refs[NSPLIT + 3:NSPLIT + 3 + NSPLIT]
        sems = refs[NSPLIT + 3 + NSPLIT:]
        wid = lax.axis_index("s") * NCORE + lax.axis_index("c")
        base = wid * TPW
        pltpu.sync_copy(dest_hbm.at[pl.ds(base, TPW)], idx_v)
        copies = [pltpu.async_copy(ys_hbm[j].at[idx_v], rows[j], sems[j])
                  for j in range(NSPLIT)]
        for j in range(NSPLIT):
            copies[j].wait()
            pltpu.sync_copy(rows[j],
                            y_hbm.at[pl.ds(base, TPW), pl.ds(j * OH, OH)])

    return _dispatch, _combine


def _gmm_body(ce_ref, xs_ref, *refs):
    w_refs = refs[:NSPLIT]
    b_refs = refs[NSPLIT:2 * NSPLIT]
    s_refs = refs[2 * NSPLIT:3 * NSPLIT]
    out_refs = refs[3 * NSPLIT:]
    xb = xs_ref[...].astype(jnp.bfloat16)
    for j in range(NSPLIT):
        w = (w_refs[j][0] + s_refs[j][...]).astype(jnp.bfloat16)
        acc = lax.dot_general(xb, w, (((1,), (1,)), ((), ())),
                              preferred_element_type=jnp.float32)
        out_refs[j][...] = acc + b_refs[j][0]


def _gmm(ce, xs, expert_W, expert_b3, static_W):
    def w_spec(j):
        return pl.BlockSpec((1, OH, D), lambda i, ce, j=j: (ce[i], j, 0))

    def s_spec(j):
        return pl.BlockSpec((OH, D), lambda i, ce, j=j: (j, 0))

    grid_spec = pltpu.PrefetchScalarGridSpec(
        num_scalar_prefetch=1,
        grid=(NCHUNK,),
        in_specs=[
            pl.BlockSpec((BT, D), lambda i, ce: (i, 0)),
            *[w_spec(j) for j in range(NSPLIT)],
            *[pl.BlockSpec((1, 1, OH),
                           lambda i, ce, j=j: (ce[i] * NSPLIT + j, 0, 0))
              for j in range(NSPLIT)],
            *[s_spec(j) for j in range(NSPLIT)],
        ],
        out_specs=tuple(
            pl.BlockSpec((BT, OH), lambda i, ce: (i, 0))
            for _ in range(NSPLIT)),
    )
    return pl.pallas_call(
        _gmm_body,
        grid_spec=grid_spec,
        out_shape=tuple(
            jax.ShapeDtypeStruct((TP, OH), jnp.float32)
            for _ in range(NSPLIT)),
    )(ce, xs, *([expert_W] * NSPLIT), *([expert_b3] * NSPLIT),
      *([static_W] * NSPLIT))


def kernel(x, w_gate, expert_W, expert_b, static_W):
    orig = x.shape[:-1]
    xf = x.reshape(-1, D)
    dest2, ce2, loss2 = _route(xf, w_gate)
    dest = dest2.reshape(T)
    ce = ce2.reshape(NCHUNK)
    dispatch, combine = _sc_kernels()
    xs = dispatch(xf, dest)
    ys = _gmm(ce, xs, expert_W,
              expert_b.reshape(E * NSPLIT, 1, OH), static_W)
    y = combine(*ys, dest)
    return (y.reshape(orig + (O,)), loss2.reshape(()))


# speedup vs baseline: 1.0980x; 1.0980x over previous
import functools

import jax
import jax.numpy as jnp
from jax import lax
from jax.experimental import pallas as pl
from jax.experimental.pallas import tpu as pltpu
from jax.experimental.pallas import tpu_sc as plsc

T = 2048
D = 768
O = 768
E = 8
BT = 256
NCHUNK = T // BT + E - 1
NSPLIT = 3
OH = O // NSPLIT
TP = NCHUNK * BT
NCORE = 2
NSUB = 16
NW = NCORE * NSUB
TPW = T // NW


def _route_body(x1_ref, x2_ref, wg_ref, dest_ref, ce_ref, loss_ref):
    xf = jnp.concatenate([x1_ref[...], x2_ref[...]], axis=1)
    wg = wg_ref[...]
    logits = lax.dot_general(xf, wg, (((1,), (0,)), ((), ())),
                             preferred_element_type=jnp.float32)
    lt = logits.T
    m = jnp.max(lt, axis=0, keepdims=True)
    eiota = lax.broadcasted_iota(jnp.int32, (E, T), 0)
    e = jnp.min(jnp.where(lt == m, eiota, E), axis=0, keepdims=True)
    oh = (eiota == e).astype(jnp.float32)
    counts = jnp.sum(oh, axis=1, keepdims=True)

    c = oh
    k = 1
    while k < T:
        c = c + jnp.concatenate(
            [jnp.zeros((E, k), jnp.float32), c[:, : T - k]], axis=1)
        k *= 2
    rank = jnp.sum(c * oh, axis=0, keepdims=True) - 1.0

    nch = jnp.floor((counts + float(BT - 1)) * (1.0 / BT))
    inc = nch
    k = 1
    while k < E:
        inc = inc + jnp.concatenate(
            [jnp.zeros((k, 1), jnp.float32), inc[: E - k, :]], axis=0)
        k *= 2
    po = (inc - nch) * float(BT)
    off_t = jnp.sum(po * oh, axis=0, keepdims=True)
    dest_ref[...] = (off_t + rank).astype(jnp.int32)

    ciota = lax.broadcasted_iota(
        jnp.int32, (E, NCHUNK), 1).astype(jnp.float32) * float(BT)
    ind = (po <= ciota).astype(jnp.int32)
    ce_ref[...] = jnp.sum(ind, axis=0, keepdims=True) - 1

    dev = counts - float(T // E)
    s = jnp.sum(dev * dev, keepdims=True)
    loss_ref[...] = s * (0.02 / float(E - 1) / float((T // E) ** 2))


def _route(xf, w_gate):
    return pl.pallas_call(
        _route_body,
        grid=(1,),
        in_specs=[
            pl.BlockSpec((T, D // 2), lambda i: (0, 0)),
            pl.BlockSpec((T, D // 2), lambda i: (0, 1)),
            pl.BlockSpec((D, E), lambda i: (0, 0)),
        ],
        out_specs=(
            pl.BlockSpec((1, T), lambda i: (0, 0)),
            pl.BlockSpec((1, NCHUNK), lambda i: (0, 0)),
            pl.BlockSpec((1, 1), lambda i: (0, 0)),
        ),
        out_shape=(
            jax.ShapeDtypeStruct((1, T), jnp.int32),
            jax.ShapeDtypeStruct((1, NCHUNK), jnp.int32),
            jax.ShapeDtypeStruct((1, 1), jnp.float32),
        ),
    )(xf, xf, w_gate)


@functools.cache
def _sc_kernels():
    mesh = plsc.VectorSubcoreMesh(core_axis_name="c", subcore_axis_name="s")

    @functools.partial(
        pl.kernel,
        mesh=mesh,
        out_type=jax.ShapeDtypeStruct((TP, D), jnp.float32),
        scratch_types=[
            pltpu.VMEM((TPW,), jnp.int32),
            pltpu.VMEM((TPW, D), jnp.float32),
            pltpu.SemaphoreType.DMA,
        ],
    )
    def _dispatch(x_hbm, dest_hbm, xs_hbm, idx_v, rows_v, sem):
        wid = lax.axis_index("s") * NCORE + lax.axis_index("c")
        base = wid * TPW
        pltpu.sync_copy(dest_hbm.at[pl.ds(base, TPW)], idx_v)
        pltpu.sync_copy(x_hbm.at[pl.ds(base, TPW)], rows_v)
        pltpu.async_copy(rows_v, xs_hbm.at[idx_v], sem).wait()

    @functools.partial(
        pl.kernel,
        mesh=mesh,
        out_type=jax.ShapeDtypeStruct((T, O), jnp.float32),
        scratch_types=[
            pltpu.VMEM((TPW,), jnp.int32),
            *[pltpu.VMEM((TPW, OH), jnp.float32) for _ in range(NSPLIT)],
            *[pltpu.SemaphoreType.DMA for _ in range(NSPLIT)],
        ],
    )
    def _combine(*refs):
        ys_hbm = refs[:NSPLIT]
        dest_hbm, y_hbm, idx_v = refs[NSPLIT:NSPLIT + 3]
        rows = refs[NSPLIT + 3:NSPLIT + 3 + NSPLIT]
        sems = refs[NSPLIT + 3 + NSPLIT:]
        wid = lax.axis_index("s") * NCORE + lax.axis_index("c")
        base = wid * TPW
        pltpu.sync_copy(dest_hbm.at[pl.ds(base, TPW)], idx_v)
        copies = [pltpu.async_copy(ys_hbm[j].at[idx_v], rows[j], sems[j])
                  for j in range(NSPLIT)]
        for j in range(NSPLIT):
            copies[j].wait()
            pltpu.sync_copy(rows[j],
                            y_hbm.at[pl.ds(base, TPW), pl.ds(j * OH, OH)])

    return _dispatch, _combine


def _gmm_body(ce_ref, xs_ref, *refs):
    w_refs = refs[:NSPLIT]
    b_refs = refs[NSPLIT:2 * NSPLIT]
    s_refs = refs[2 * NSPLIT:3 * NSPLIT]
    out_refs = refs[3 * NSPLIT:]
    xb = xs_ref[...].astype(jnp.bfloat16)
    for j in range(NSPLIT):
        w = (w_refs[j][0] + s_refs[j][...]).astype(jnp.bfloat16)
        acc = lax.dot_general(xb, w, (((1,), (1,)), ((), ())),
                              preferred_element_type=jnp.float32)
        out_refs[j][...] = acc + b_refs[j][0]


def _gmm(ce, xs, expert_W, expert_b3, static_W):
    def w_spec(j):
        return pl.BlockSpec((1, OH, D), lambda i, ce, j=j: (ce[i], j, 0))

    def s_spec(j):
        return pl.BlockSpec((OH, D), lambda i, ce, j=j: (j, 0))

    grid_spec = pltpu.PrefetchScalarGridSpec(
        num_scalar_prefetch=1,
        grid=(NCHUNK,),
        in_specs=[
            pl.BlockSpec((BT, D), lambda i, ce: (i, 0)),
            *[w_spec(j) for j in range(NSPLIT)],
            *[pl.BlockSpec((1, 1, OH),
                           lambda i, ce, j=j: (ce[i] * NSPLIT + j, 0, 0))
              for j in range(NSPLIT)],
            *[s_spec(j) for j in range(NSPLIT)],
        ],
        out_specs=tuple(
            pl.BlockSpec((BT, OH), lambda i, ce: (i, 0))
            for _ in range(NSPLIT)),
    )
    return pl.pallas_call(
        _gmm_body,
        grid_spec=grid_spec,
        out_shape=tuple(
            jax.ShapeDtypeStruct((TP, OH), jnp.float32)
            for _ in range(NSPLIT)),
    )(ce, xs, *([expert_W] * NSPLIT), *([expert_b3] * NSPLIT),
      *([static_W] * NSPLIT))


def kernel(x, w_gate, expert_W, expert_b, static_W):
    orig = x.shape[:-1]
    xf = x.reshape(-1, D)
    dest2, ce2, loss2 = _route(xf, w_gate)
    dest = dest2.reshape(T)
    ce = ce2.reshape(NCHUNK)
    dispatch, combine = _sc_kernels()
    xs = dispatch(xf, dest)
    ys = _gmm(ce, xs, expert_W,
              expert_b.reshape(E * NSPLIT, 1, OH), static_W)
    y = combine(*ys, dest)
    return (y.reshape(orig + (O,)), loss2.reshape(()))


# speedup vs baseline: 1.1196x; 1.0197x over previous
import functools

import jax
import jax.numpy as jnp
from jax import lax
from jax.experimental import pallas as pl
from jax.experimental.pallas import tpu as pltpu
from jax.experimental.pallas import tpu_sc as plsc

T = 2048
D = 768
O = 768
E = 8
BT = 256
NT = T // BT
NCHUNK = NT + E - 1
NSPLIT = 3
OH = O // NSPLIT
TP = T
NCORE = 2
NSUB = 16
NW = NCORE * NSUB
TPW = T // NW


def _route_body(x1_ref, x2_ref, wg_ref, dest_ref, ce_ref, loss_ref):
    xf = jnp.concatenate([x1_ref[...], x2_ref[...]], axis=1)
    wg = wg_ref[...]
    logits = lax.dot_general(xf, wg, (((1,), (0,)), ((), ())),
                             preferred_element_type=jnp.float32)
    lt = logits.T
    m = jnp.max(lt, axis=0, keepdims=True)
    eiota = lax.broadcasted_iota(jnp.int32, (E, T), 0)
    e = jnp.min(jnp.where(lt == m, eiota, E), axis=0, keepdims=True)
    oh = (eiota == e).astype(jnp.float32)
    counts = jnp.sum(oh, axis=1, keepdims=True)

    c = oh
    k = 1
    while k < T:
        c = c + jnp.concatenate(
            [jnp.zeros((E, k), jnp.float32), c[:, : T - k]], axis=1)
        k *= 2
    rank = jnp.sum(c * oh, axis=0, keepdims=True) - 1.0

    incc = counts
    k = 1
    while k < E:
        incc = incc + jnp.concatenate(
            [jnp.zeros((k, 1), jnp.float32), incc[: E - k, :]], axis=0)
        k *= 2
    o_ex = incc - counts
    off_t = jnp.sum(o_ex * oh, axis=0, keepdims=True)
    dest_ref[...] = (off_t + rank).astype(jnp.int32)

    tpos = lax.broadcasted_iota(
        jnp.int32, (E, NT), 1).astype(jnp.float32) * float(BT)
    f_t = jnp.sum((o_ex <= tpos).astype(jnp.float32),
                  axis=0, keepdims=True) - 1.0
    l_t = jnp.sum((o_ex <= tpos + float(BT - 1)).astype(jnp.float32),
                  axis=0, keepdims=True) - 1.0
    nit = l_t - f_t + 1.0
    sit = nit
    k = 1
    while k < NT:
        sit = sit + jnp.concatenate(
            [jnp.zeros((1, k), jnp.float32), sit[:, : NT - k]], axis=1)
        k *= 2
    s_ex = sit - nit
    s_col = s_ex.T
    wit = lax.broadcasted_iota(
        jnp.int32, (NT, NCHUNK), 1).astype(jnp.float32)
    t_it = jnp.sum((s_col <= wit).astype(jnp.float32),
                   axis=0, keepdims=True) - 1.0
    oh_t = (lax.broadcasted_iota(jnp.int32, (NT, NCHUNK), 0)
            .astype(jnp.float32) == t_it).astype(jnp.float32)
    f_at = jnp.sum(f_t.T * oh_t, axis=0, keepdims=True)
    s_at = jnp.sum(s_col * oh_t, axis=0, keepdims=True)
    w_row = lax.broadcasted_iota(
        jnp.int32, (1, NCHUNK), 1).astype(jnp.float32)
    e_it = jnp.minimum(f_at + (w_row - s_at), float(E - 1))
    oh_e = (lax.broadcasted_iota(jnp.int32, (E, NCHUNK), 0)
            .astype(jnp.float32) == e_it).astype(jnp.float32)
    o_at = jnp.sum(o_ex * oh_e, axis=0, keepdims=True)
    on_at = jnp.sum(incc * oh_e, axis=0, keepdims=True)
    tb = t_it * float(BT)
    lo = jnp.maximum(o_at, tb) - tb
    hi = jnp.minimum(on_at, tb + float(BT)) - tb
    wk = jnp.concatenate([t_it, e_it, lo, hi], axis=0)
    ce_ref[...] = wk.astype(jnp.int32)

    dev = counts - float(T // E)
    s = jnp.sum(dev * dev, keepdims=True)
    loss_ref[...] = s * (0.02 / float(E - 1) / float((T // E) ** 2))


def _route(xf, w_gate):
    return pl.pallas_call(
        _route_body,
        grid=(1,),
        in_specs=[
            pl.BlockSpec((T, D // 2), lambda i: (0, 0)),
            pl.BlockSpec((T, D // 2), lambda i: (0, 1)),
            pl.BlockSpec((D, E), lambda i: (0, 0)),
        ],
        out_specs=(
            pl.BlockSpec((1, T), lambda i: (0, 0)),
            pl.BlockSpec((4, NCHUNK), lambda i: (0, 0)),
            pl.BlockSpec((1, 1), lambda i: (0, 0)),
        ),
        out_shape=(
            jax.ShapeDtypeStruct((1, T), jnp.int32),
            jax.ShapeDtypeStruct((4, NCHUNK), jnp.int32),
            jax.ShapeDtypeStruct((1, 1), jnp.float32),
        ),
    )(xf, xf, w_gate)


@functools.cache
def _sc_kernels():
    mesh = plsc.VectorSubcoreMesh(core_axis_name="c", subcore_axis_name="s")

    @functools.partial(
        pl.kernel,
        mesh=mesh,
        out_type=jax.ShapeDtypeStruct((TP, D), jnp.float32),
        scratch_types=[
            pltpu.VMEM((TPW,), jnp.int32),
            pltpu.VMEM((TPW, D), jnp.float32),
            pltpu.SemaphoreType.DMA,
        ],
    )
    def _dispatch(x_hbm, dest_hbm, xs_hbm, idx_v, rows_v, sem):
        wid = lax.axis_index("s") * NCORE + lax.axis_index("c")
        base = wid * TPW
        pltpu.sync_copy(dest_hbm.at[pl.ds(base, TPW)], idx_v)
        pltpu.sync_copy(x_hbm.at[pl.ds(base, TPW)], rows_v)
        pltpu.async_copy(rows_v, xs_hbm.at[idx_v], sem).wait()

    @functools.partial(
        pl.kernel,
        mesh=mesh,
        out_type=jax.ShapeDtypeStruct((T, O), jnp.float32),
        scratch_types=[
            pltpu.VMEM((TPW,), jnp.int32),
            *[pltpu.VMEM((TPW, OH), jnp.float32) for _ in range(NSPLIT)],
            *[pltpu.SemaphoreType.DMA for _ in range(NSPLIT)],
        ],
    )
    def _combine(*refs):
        ys_hbm = refs[:NSPLIT]
        dest_hbm, y_hbm, idx_v = refs[NSPLIT:NSPLIT + 3]
        rows = refs[NSPLIT + 3:NSPLIT + 3 + NSPLIT]
        sems = refs[NSPLIT + 3 + NSPLIT:]
        wid = lax.axis_index("s") * NCORE + lax.axis_index("c")
        base = wid * TPW
        pltpu.sync_copy(dest_hbm.at[pl.ds(base, TPW)], idx_v)
        copies = [pltpu.async_copy(ys_hbm[j].at[idx_v], rows[j], sems[j])
                  for j in range(NSPLIT)]
        for j in range(NSPLIT):
            copies[j].wait()
            pltpu.sync_copy(rows[j],
                            y_hbm.at[pl.ds(base, TPW), pl.ds(j * OH, OH)])

    return _dispatch, _combine


def _gmm_body(wk_ref, xs_ref, *refs):
    w_refs = refs[:NSPLIT]
    b_refs = refs[NSPLIT:2 * NSPLIT]
    s_refs = refs[2 * NSPLIT:3 * NSPLIT]
    out_refs = refs[3 * NSPLIT:]
    i = pl.program_id(0)
    lo = wk_ref[2, i]
    hi = wk_ref[3, i]
    riota = lax.broadcasted_iota(jnp.int32, (BT, 1), 0)
    mask = jnp.logical_and(riota >= lo, riota < hi)
    xb = xs_ref[...].astype(jnp.bfloat16)
    for j in range(NSPLIT):
        w = (w_refs[j][0] + s_refs[j][...]).astype(jnp.bfloat16)
        acc = lax.dot_general(xb, w, (((1,), (1,)), ((), ())),
                              preferred_element_type=jnp.float32)
        out_refs[j][...] = jnp.where(mask, acc + b_refs[j][0],
                                     out_refs[j][...])


def _gmm(wk, xs, expert_W, expert_b3, static_W):
    def w_spec(j):
        return pl.BlockSpec((1, OH, D), lambda i, wk, j=j: (wk[1, i], j, 0))

    def s_spec(j):
        return pl.BlockSpec((OH, D), lambda i, wk, j=j: (j, 0))

    grid_spec = pltpu.PrefetchScalarGridSpec(
        num_scalar_prefetch=1,
        grid=(NCHUNK,),
        in_specs=[
            pl.BlockSpec((BT, D), lambda i, wk: (wk[0, i], 0)),
            *[w_spec(j) for j in range(NSPLIT)],
            *[pl.BlockSpec((1, 1, OH),
                           lambda i, wk, j=j: (wk[1, i] * NSPLIT + j, 0, 0))
              for j in range(NSPLIT)],
            *[s_spec(j) for j in range(NSPLIT)],
        ],
        out_specs=tuple(
            pl.BlockSpec((BT, OH), lambda i, wk, j=j: (wk[0, i], 0))
            for j in range(NSPLIT)),
    )
    return pl.pallas_call(
        _gmm_body,
        grid_spec=grid_spec,
        out_shape=tuple(
            jax.ShapeDtypeStruct((TP, OH), jnp.float32)
            for _ in range(NSPLIT)),
    )(wk, xs, *([expert_W] * NSPLIT), *([expert_b3] * NSPLIT),
      *([static_W] * NSPLIT))


def kernel(x, w_gate, expert_W, expert_b, static_W):
    orig = x.shape[:-1]
    xf = x.reshape(-1, D)
    dest2, wk, loss2 = _route(xf, w_gate)
    dest = dest2.reshape(T)
    dispatch, combine = _sc_kernels()
    xs = dispatch(xf, dest)
    ys = _gmm(wk, xs, expert_W,
              expert_b.reshape(E * NSPLIT, 1, OH), static_W)
    y = combine(*ys, dest)
    return (y.reshape(orig + (O,)), loss2.reshape(()))
